# baseline (speedup 1.0000x reference)
import functools

import jax
import jax.numpy as jnp
import numpy as np
from jax import lax
from jax.experimental import pallas as pl
from jax.experimental.pallas import tpu as pltpu
from jax.experimental.pallas import tpu_sc as plsc

LANES = 16
G = 128
NBUF = 4
LOOK = 2
PITCH = 129


def _emb_sc(table, xt, n_b1, D):
    scale = float(D) ** 0.5
    info = plsc.get_sparse_core_info()
    NC, NS = info.num_cores, info.num_subcores
    NW = NC * NS
    DB = D // 8

    mesh = plsc.VectorSubcoreMesh(core_axis_name="c", subcore_axis_name="s")

    @functools.partial(
        pl.kernel,
        mesh=mesh,
        compiler_params=pltpu.CompilerParams(
            use_tc_tiling_on_sc=False, needs_layout_passes=False),
        out_type=jax.ShapeDtypeStruct((n_b1, DB, NW, 8, G), jnp.float32),
        scratch_types=[
            pltpu.VMEM((n_b1, G), jnp.int32),
            pltpu.VMEM((NBUF, G, D), jnp.float32),
            pltpu.VMEM((NBUF, D, PITCH), jnp.float32),
            [pltpu.SemaphoreType.DMA] * NBUF,
            [pltpu.SemaphoreType.DMA] * NBUF,
        ],
    )
    def emb(table_hbm, xt_hbm, out_hbm, idx_v, rows_v, tbuf, gsems, wsems):
        c_ax = lax.axis_index("c")
        s_ax = lax.axis_index("s")
        w = s_ax * NC + c_ax
        pltpu.sync_copy(xt_hbm.at[:, pl.ds(w * G, G)], idx_v)

        def fire_gather(b, g):
            pltpu.async_copy(table_hbm.at[idx_v.at[g]], rows_v.at[b], gsems[b])

        def wait_gather(b, g):
            pltpu.make_async_copy(
                table_hbm.at[idx_v.at[g]], rows_v.at[b], gsems[b]).wait()

        def fire_writes(b, g):
            for k in range(DB):
                pltpu.async_copy(
                    tbuf.at[b, pl.ds(k * 8, 8), pl.ds(0, G)],
                    out_hbm.at[g, k, w], wsems[b])

        def wait_writes(b, g):
            for k in range(DB):
                pltpu.make_async_copy(
                    tbuf.at[b, pl.ds(k * 8, 8), pl.ds(0, G)],
                    out_hbm.at[g, k, w], wsems[b]).wait()

        base_iota = lax.iota(jnp.int32, LANES)
        row_idx = [base_iota + d0 for d0 in range(0, D, LANES)]

        for b in range(LOOK):
            fire_gather(b, b)

        def outer(o, carry):
            for b in range(NBUF):
                g = o * NBUF + b
                bf = (b + LOOK) % NBUF
                pf = g + LOOK

                @pl.when(pf < n_b1)
                def _():
                    @pl.when(g >= LOOK)
                    def _():
                        wait_writes(bf, g - LOOK)
                    fire_gather(bf, pf)

                wait_gather(b, g)

                @plsc.parallel_loop(0, G, unroll=16)
                def col_body(c):
                    col = jnp.full((LANES,), c, dtype=jnp.int32)
                    for j in range(D // LANES):
                        vals = rows_v[b, c, pl.ds(j * LANES, LANES)] * scale
                        plsc.store_scatter(
                            tbuf.at[b], [row_idx[j], col], vals)
                fire_writes(b, g)
            return carry

        lax.fori_loop(0, n_b1 // NBUF, outer, 0)

        for b in range(NBUF):
            wait_writes(b, n_b1 - NBUF + b)

    return emb(table, xt)


def kernel(x, table):
    B0, B1 = x.shape
    D = table.shape[1]
    xt = jnp.swapaxes(x, 0, 1).astype(jnp.int32)
    out5 = _emb_sc(table, xt, B1, D)
    return out5.transpose(2, 4, 0, 1, 3).reshape(B0, B1, D)


# speedup vs baseline: 1.0472x; 1.0472x over previous
import functools

import jax
import jax.numpy as jnp
import numpy as np
from jax import lax
from jax.experimental import pallas as pl
from jax.experimental.pallas import tpu as pltpu
from jax.experimental.pallas import tpu_sc as plsc

LANES = 16
G = 128
NBUF = 4
LOOK = 2
PITCH = 129


def _emb_sc(table, xt, n_b1, D):
    scale = float(D) ** 0.5
    info = plsc.get_sparse_core_info()
    NC, NS = info.num_cores, info.num_subcores
    NW = NC * NS
    DB = D // 8

    mesh = plsc.VectorSubcoreMesh(core_axis_name="c", subcore_axis_name="s")

    @functools.partial(
        pl.kernel,
        mesh=mesh,
        compiler_params=pltpu.CompilerParams(
            use_tc_tiling_on_sc=False, needs_layout_passes=False),
        out_type=jax.ShapeDtypeStruct((n_b1, DB, NW, 8, G), jnp.float32),
        scratch_types=[
            pltpu.VMEM((n_b1, G), jnp.int32),
            pltpu.VMEM((NBUF, G, D), jnp.float32),
            pltpu.VMEM((NBUF, D, PITCH), jnp.float32),
            [pltpu.SemaphoreType.DMA] * NBUF,
            [pltpu.SemaphoreType.DMA] * NBUF,
        ],
    )
    def emb(table_hbm, xt_hbm, out_hbm, idx_v, rows_v, tbuf, gsems, wsems):
        c_ax = lax.axis_index("c")
        s_ax = lax.axis_index("s")
        w = s_ax * NC + c_ax
        pltpu.sync_copy(xt_hbm.at[:, pl.ds(w * G, G)], idx_v)

        def fire_gather(b, g):
            pltpu.async_copy(table_hbm.at[idx_v.at[g]], rows_v.at[b], gsems[b])

        def wait_gather(b, g):
            pltpu.make_async_copy(
                table_hbm.at[idx_v.at[g]], rows_v.at[b], gsems[b]).wait()

        def fire_writes(b, g):
            for k in range(DB):
                pltpu.async_copy(
                    tbuf.at[b, pl.ds(k * 8, 8), pl.ds(0, G)],
                    out_hbm.at[g, k, w], wsems[b])

        def wait_writes(b, g):
            for k in range(DB):
                pltpu.make_async_copy(
                    tbuf.at[b, pl.ds(k * 8, 8), pl.ds(0, G)],
                    out_hbm.at[g, k, w], wsems[b]).wait()

        base_iota = lax.iota(jnp.int32, LANES)
        row_idx = [base_iota + d0 for d0 in range(0, D, LANES)]

        for b in range(LOOK):
            fire_gather(b, b)

        def outer(o, carry):
            for b in range(NBUF):
                g = o * NBUF + b
                bf = (b + LOOK) % NBUF
                pf = g + LOOK

                @pl.when(pf < n_b1)
                def _():
                    @pl.when(g >= LOOK)
                    def _():
                        wait_writes(bf, g - LOOK)
                    fire_gather(bf, pf)

                wait_gather(b, g)

                @plsc.parallel_loop(0, G, unroll=4)
                def col_body(c):
                    col = jnp.full((LANES,), c, dtype=jnp.int32)
                    for j in range(D // LANES):
                        vals = rows_v[b, c, pl.ds(j * LANES, LANES)] * scale
                        plsc.store_scatter(
                            tbuf.at[b], [row_idx[j], col], vals)
                fire_writes(b, g)
            return carry

        lax.fori_loop(0, n_b1 // NBUF, outer, 0)

        for b in range(NBUF):
            wait_writes(b, n_b1 - NBUF + b)

    return emb(table, xt)


def kernel(x, table):
    B0, B1 = x.shape
    D = table.shape[1]
    xt = jnp.swapaxes(x, 0, 1).astype(jnp.int32)
    out5 = _emb_sc(table, xt, B1, D)
    return out5.transpose(2, 4, 0, 1, 3).reshape(B0, B1, D)


# speedup vs baseline: 1.0474x; 1.0002x over previous
import functools

import jax
import jax.numpy as jnp
import numpy as np
from jax import lax
from jax.experimental import pallas as pl
from jax.experimental.pallas import tpu as pltpu
from jax.experimental.pallas import tpu_sc as plsc

LANES = 16
G = 128
NBUF = 4
LOOK = 2
PITCH = 129


def _emb_sc(table, xt, n_b1, D):
    scale = float(D) ** 0.5
    info = plsc.get_sparse_core_info()
    NC, NS = info.num_cores, info.num_subcores
    NW = NC * NS
    DB = D // 8

    mesh = plsc.VectorSubcoreMesh(core_axis_name="c", subcore_axis_name="s")

    @functools.partial(
        pl.kernel,
        mesh=mesh,
        compiler_params=pltpu.CompilerParams(
            use_tc_tiling_on_sc=False, needs_layout_passes=False),
        out_type=jax.ShapeDtypeStruct((n_b1, DB, NW, 8, G), jnp.float32),
        scratch_types=[
            pltpu.VMEM((n_b1, G), jnp.int32),
            pltpu.VMEM((NBUF, G, D), jnp.float32),
            pltpu.VMEM((NBUF, D, PITCH), jnp.float32),
            [pltpu.SemaphoreType.DMA] * NBUF,
            [pltpu.SemaphoreType.DMA] * NBUF,
        ],
    )
    def emb(table_hbm, xt_hbm, out_hbm, idx_v, rows_v, tbuf, gsems, wsems):
        c_ax = lax.axis_index("c")
        s_ax = lax.axis_index("s")
        w = s_ax * NC + c_ax
        pltpu.sync_copy(xt_hbm.at[:, pl.ds(w * G, G)], idx_v)

        def fire_gather(b, g):
            pltpu.async_copy(table_hbm.at[idx_v.at[g]], rows_v.at[b], gsems[b])

        def wait_gather(b, g):
            pltpu.make_async_copy(
                table_hbm.at[idx_v.at[g]], rows_v.at[b], gsems[b]).wait()

        def fire_writes(b, g):
            for k in range(DB):
                pltpu.async_copy(
                    tbuf.at[b, pl.ds(k * 8, 8), pl.ds(0, G)],
                    out_hbm.at[g, k, w], wsems[b])

        def wait_writes(b, g):
            for k in range(DB):
                pltpu.make_async_copy(
                    tbuf.at[b, pl.ds(k * 8, 8), pl.ds(0, G)],
                    out_hbm.at[g, k, w], wsems[b]).wait()

        base_iota = lax.iota(jnp.int32, LANES)
        row_idx = [base_iota + d0 for d0 in range(0, D, LANES)]

        for b in range(LOOK):
            fire_gather(b, b)

        def outer(o, carry):
            for b in range(NBUF):
                g = o * NBUF + b
                bf = (b + LOOK) % NBUF
                pf = g + LOOK

                @pl.when(pf < n_b1)
                def _():
                    @pl.when(g >= LOOK)
                    def _():
                        wait_writes(bf, g - LOOK)
                    fire_gather(bf, pf)

                wait_gather(b, g)

                @plsc.parallel_loop(0, G, unroll=2)
                def col_body(c):
                    col = jnp.full((LANES,), c, dtype=jnp.int32)
                    for j in range(D // LANES):
                        vals = rows_v[b, c, pl.ds(j * LANES, LANES)] * scale
                        plsc.store_scatter(
                            tbuf.at[b], [row_idx[j], col], vals)
                fire_writes(b, g)
            return carry

        lax.fori_loop(0, n_b1 // NBUF, outer, 0)

        for b in range(NBUF):
            wait_writes(b, n_b1 - NBUF + b)

    return emb(table, xt)


def kernel(x, table):
    B0, B1 = x.shape
    D = table.shape[1]
    xt = jnp.swapaxes(x, 0, 1).astype(jnp.int32)
    out5 = _emb_sc(table, xt, B1, D)
    return out5.transpose(2, 4, 0, 1, 3).reshape(B0, B1, D)


# speedup vs baseline: 1.0480x; 1.0006x over previous
import functools

import jax
import jax.numpy as jnp
import numpy as np
from jax import lax
from jax.experimental import pallas as pl
from jax.experimental.pallas import tpu as pltpu
from jax.experimental.pallas import tpu_sc as plsc

LANES = 16
G = 128
NBUF = 4
LOOK = 2
PITCH = 129


def _emb_sc(table, xt, n_b1, D):
    scale = float(D) ** 0.5
    info = plsc.get_sparse_core_info()
    NC, NS = info.num_cores, info.num_subcores
    NW = NC * NS
    DB = D // 8

    mesh = plsc.VectorSubcoreMesh(core_axis_name="c", subcore_axis_name="s")

    @functools.partial(
        pl.kernel,
        mesh=mesh,
        compiler_params=pltpu.CompilerParams(
            use_tc_tiling_on_sc=False, needs_layout_passes=False),
        out_type=jax.ShapeDtypeStruct((n_b1, DB, NW, 8, G), jnp.float32),
        scratch_types=[
            pltpu.VMEM((n_b1, G), jnp.int32),
            pltpu.VMEM((NBUF, G, D), jnp.float32),
            pltpu.VMEM((NBUF, D, PITCH), jnp.float32),
            [pltpu.SemaphoreType.DMA] * NBUF,
            [pltpu.SemaphoreType.DMA] * NBUF,
        ],
    )
    def emb(table_hbm, xt_hbm, out_hbm, idx_v, rows_v, tbuf, gsems, wsems):
        c_ax = lax.axis_index("c")
        s_ax = lax.axis_index("s")
        w = s_ax * NC + c_ax
        pltpu.sync_copy(xt_hbm.at[:, pl.ds(w * G, G)], idx_v)

        def fire_gather(b, g):
            pltpu.async_copy(table_hbm.at[idx_v.at[g]], rows_v.at[b], gsems[b])

        def wait_gather(b, g):
            pltpu.make_async_copy(
                table_hbm.at[idx_v.at[g]], rows_v.at[b], gsems[b]).wait()

        def fire_writes(b, g):
            for k in range(DB):
                pltpu.async_copy(
                    tbuf.at[b, pl.ds(k * 8, 8), pl.ds(0, G)],
                    out_hbm.at[g, k, w], wsems[b])

        def wait_writes(b, g):
            for k in range(DB):
                pltpu.make_async_copy(
                    tbuf.at[b, pl.ds(k * 8, 8), pl.ds(0, G)],
                    out_hbm.at[g, k, w], wsems[b]).wait()

        base_iota = lax.iota(jnp.int32, LANES)
        row_idx = [base_iota + d0 for d0 in range(0, D, LANES)]

        for b in range(LOOK):
            fire_gather(b, b)

        def outer(o, carry):
            for b in range(NBUF):
                g = o * NBUF + b
                bf = (b + LOOK) % NBUF
                pf = g + LOOK

                @pl.when(pf < n_b1)
                def _():
                    @pl.when(g >= LOOK)
                    def _():
                        wait_writes(bf, g - LOOK)
                    fire_gather(bf, pf)

                wait_gather(b, g)

                @plsc.parallel_loop(0, G, unroll=4)
                def col_body(c):
                    col = jnp.full((LANES,), c, dtype=jnp.int32)
                    for j in range(D // LANES):
                        vals = rows_v[b, c, pl.ds(j * LANES, LANES)] * scale
                        plsc.store_scatter(
                            tbuf.at[b], [row_idx[j], col], vals)
                fire_writes(b, g)
            return carry

        lax.fori_loop(0, n_b1 // NBUF, outer, 0)

        for b in range(NBUF):
            wait_writes(b, n_b1 - NBUF + b)

    return emb(table, xt)


def kernel(x, table):
    B0, B1 = x.shape
    D = table.shape[1]
    xt = jnp.swapaxes(x, 0, 1).astype(jnp.int32)
    out5 = _emb_sc(table, xt, B1, D)
    return out5.transpose(2, 4, 0, 1, 3).reshape(B0, B1, D)
